# baseline (speedup 1.0000x reference)
import functools
import numpy as np
import jax
import jax.numpy as jnp
from jax import lax
from jax.experimental import pallas as pl
from jax.experimental.pallas import tpu as pltpu
from jax.experimental.pallas import tpu_sc as plsc

DIM = 32
SCALE = np.sqrt(np.float32(DIM)).astype(np.float32)
NC, NS = 2, 16
NW = NC * NS
SLAB = 2
NBUF = 4
GATHER_AHEAD = 3


@functools.lru_cache(maxsize=None)
def _make(B1, B2):
    rows_w = B1 // NW
    n_chunks = rows_w // SLAB
    n_groups = n_chunks // NBUF
    kstream = (SLAB * B2) // 100
    assert (SLAB * B2) % 100 == 0 and n_chunks % NBUF == 0
    mesh = plsc.VectorSubcoreMesh(
        core_axis_name="c", subcore_axis_name="s",
        num_cores=NC, num_subcores=NS)

    @functools.partial(
        pl.kernel,
        out_type=jax.ShapeDtypeStruct((B1, B2, DIM), jnp.float32),
        mesh=mesh,
        scratch_types=(
            [pltpu.VMEM((rows_w * B2 // 100, 100), jnp.int32)]
            + [pltpu.VMEM((SLAB, B2, DIM), jnp.float32)] * NBUF
            + [pltpu.SemaphoreType.DMA] * (2 * NBUF)
        ),
        compiler_params=pltpu.CompilerParams(use_tc_tiling_on_sc=False),
    )
    def emb_kernel(table_hbm, idx_hbm, out_hbm, idx_v, *scratch):
        bufs = scratch[:NBUF]
        gsems = scratch[NBUF:2 * NBUF]
        ssems = scratch[2 * NBUF:]
        wid = lax.axis_index("s") * NC + lax.axis_index("c")
        row0 = wid * rows_w

        def fire_gather(c, b):
            for j in range(kstream):
                pltpu.async_copy(
                    table_hbm.at[idx_v.at[c * kstream + j]],
                    bufs[b].at[(100 * j) // B2, pl.ds((100 * j) % B2, 100)],
                    gsems[b])

        def wait_gather(b):
            pltpu.make_async_copy(
                out_hbm.at[pl.ds(0, SLAB)], bufs[b], gsems[b]).wait()

        def fire_scatter(c, b):
            pltpu.async_copy(
                bufs[b], out_hbm.at[pl.ds(row0 + c * SLAB, SLAB)], ssems[b])

        def wait_scatter(b):
            pltpu.make_async_copy(
                bufs[b], out_hbm.at[pl.ds(0, SLAB)], ssems[b]).wait()

        def scale(b):
            buf = bufs[b]

            @plsc.parallel_loop(0, B2, step=1, unroll=8)
            def _scale(r):
                for s in range(SLAB):
                    buf[s, r, pl.ds(0, 16)] = buf[s, r, pl.ds(0, 16)] * SCALE
                    buf[s, r, pl.ds(16, 16)] = buf[s, r, pl.ds(16, 16)] * SCALE

        pltpu.sync_copy(idx_hbm.at[wid], idx_v)

        for c in range(GATHER_AHEAD):
            fire_gather(c, c % NBUF)

        @pl.loop(0, n_groups)
        def _group(g):
            for i in range(NBUF):
                c = g * NBUF + i
                wait_gather(i)
                scale(i)
                fire_scatter(c, i)
                inext = (i + GATHER_AHEAD) % NBUF

                @pl.when(c + GATHER_AHEAD < n_chunks)
                def _():
                    @pl.when(c + GATHER_AHEAD >= NBUF)
                    def _():
                        wait_scatter(inext)
                    fire_gather(c + GATHER_AHEAD, inext)

        for c in range(n_chunks - NBUF, n_chunks):
            wait_scatter(c % NBUF)

    return emb_kernel


def kernel(x, table):
    B1, B2 = x.shape
    idx = x.reshape(NW, (B1 // NW) * B2 // 100, 100).astype(jnp.int32)
    return _make(B1, B2)(table, idx)
